# baseline (device time: 243280 ns/iter reference)
import jax
import jax.numpy as jnp
from jax import lax
from jax.experimental import pallas as pl
from jax.experimental.pallas import tpu as pltpu

WMAX = 512
SIZES = [256] + [512] * 7 + [128, 128]
OFFS = [sum(SIZES[:i]) for i in range(len(SIZES))]
N_P = 4
K_SLAB = 256


def kernel(x, dy):
    k_per, m = x.shape
    _, f = dy.shape
    m_out = m // 2
    half_f = f // 2
    n_chunks = len(SIZES)
    assert sum(SIZES) == half_f
    n_slabs = k_per // K_SLAB

    def body(x_ref, dy_ref, out_ref, xb, xl0, xl1, dyv0, dyv1, dyb,
             p0, p1, p2, p3, recv_flat,
             x_sems, dy_sems, st_sems, sx, rx, sy, ry):
        my_x = lax.axis_index("x")
        my_y = lax.axis_index("y")
        other_x = 1 - my_x
        other_y = 1 - my_y
        my_row0 = my_x * m_out
        other_row0 = other_x * m_out

        xl = [xl0, xl1]
        dyv = [dyv0, dyv1]
        pb = [p0, p1, p2, p3]
        ld_objs = [None] * n_chunks
        st_objs = [None] * n_chunks
        rdx_objs = [None] * n_chunks
        rdy_objs = [None] * n_chunks

        def start_load(c):
            ld = pltpu.make_async_copy(
                dy_ref.at[:, pl.ds(my_y * half_f + OFFS[c], SIZES[c])],
                dyv[c % 2].at[:, pl.ds(0, SIZES[c])], dy_sems.at[c % 2])
            ld.start()
            ld_objs[c] = ld

        def finish_chunk(c):
            w = SIZES[c]
            gcol = my_y * half_f + OFFS[c]
            p = pb[c % N_P]
            rdx_objs[c].wait_recv()
            p[pl.ds(my_row0, m_out), pl.ds(0, w)] = (
                p[pl.ds(my_row0, m_out), pl.ds(0, w)]
                + recv_flat[:, pl.ds(OFFS[c], w)])
            rdma_y = pltpu.make_async_remote_copy(
                src_ref=p.at[pl.ds(my_row0, m_out), pl.ds(0, w)],
                dst_ref=out_ref.at[:, pl.ds(gcol, w)],
                send_sem=sy.at[c],
                recv_sem=ry.at[c],
                device_id=(my_x, other_y),
                device_id_type=pl.DeviceIdType.MESH,
            )
            rdma_y.start()
            rdy_objs[c] = rdma_y
            st = pltpu.make_async_copy(
                p.at[pl.ds(my_row0, m_out), pl.ds(0, w)],
                out_ref.at[:, pl.ds(gcol, w)], st_sems.at[c % N_P])
            st.start()
            st_objs[c] = st

        start_load(0)
        x_lds = []
        for i in range(min(2, n_slabs)):
            ld = pltpu.make_async_copy(
                x_ref.at[pl.ds(i * K_SLAB, K_SLAB), :], xl[i % 2],
                x_sems.at[i % 2])
            ld.start()
            x_lds.append(ld)
        for i in range(n_slabs):
            x_lds[i].wait()
            if i + 2 < n_slabs:
                ld = pltpu.make_async_copy(
                    x_ref.at[pl.ds((i + 2) * K_SLAB, K_SLAB), :],
                    xl[(i + 2) % 2], x_sems.at[(i + 2) % 2])
                ld.start()
                x_lds.append(ld)
            xb[pl.ds(i * K_SLAB, K_SLAB), :] = (
                xl[i % 2][...].astype(jnp.bfloat16))

        barrier = pltpu.get_barrier_semaphore()
        pl.semaphore_signal(barrier, inc=1, device_id=(other_x, my_y),
                            device_id_type=pl.DeviceIdType.MESH)
        pl.semaphore_signal(barrier, inc=1, device_id=(my_x, other_y),
                            device_id_type=pl.DeviceIdType.MESH)
        pl.semaphore_wait(barrier, 2)

        for c in range(n_chunks):
            w = SIZES[c]
            p = pb[c % N_P]
            ld_objs[c].wait()
            dyb[:, pl.ds(0, w)] = (
                dyv[c % 2][:, pl.ds(0, w)].astype(jnp.bfloat16))
            if c + 1 < n_chunks:
                start_load(c + 1)

            if c >= N_P:
                rdx_objs[c - N_P].wait_send()
                rdy_objs[c - N_P].wait_send()
                st_objs[c - N_P].wait()

            p[:, pl.ds(0, w)] = lax.dot_general(
                xb[...], dyb[:, pl.ds(0, w)], (((0,), (0,)), ((), ())),
                preferred_element_type=jnp.float32)

            rdma_x = pltpu.make_async_remote_copy(
                src_ref=p.at[pl.ds(other_row0, m_out), pl.ds(0, w)],
                dst_ref=recv_flat.at[:, pl.ds(OFFS[c], w)],
                send_sem=sx.at[c],
                recv_sem=rx.at[c],
                device_id=(other_x, my_y),
                device_id_type=pl.DeviceIdType.MESH,
            )
            rdma_x.start()
            rdx_objs[c] = rdma_x

            if c >= 1:
                finish_chunk(c - 1)

        finish_chunk(n_chunks - 1)

        for c in range(max(n_chunks - N_P, 0), n_chunks):
            rdx_objs[c].wait_send()
            rdy_objs[c].wait_send()
            st_objs[c].wait()
        for c in range(n_chunks):
            rdy_objs[c].wait_recv()

    return pl.pallas_call(
        body,
        out_shape=jax.ShapeDtypeStruct((m_out, f), jnp.float32),
        in_specs=[
            pl.BlockSpec(memory_space=pl.MemorySpace.ANY),
            pl.BlockSpec(memory_space=pl.MemorySpace.ANY),
        ],
        out_specs=pl.BlockSpec(memory_space=pl.MemorySpace.ANY),
        scratch_shapes=[
            pltpu.VMEM((k_per, m), jnp.bfloat16),
            pltpu.VMEM((K_SLAB, m), jnp.float32),
            pltpu.VMEM((K_SLAB, m), jnp.float32),
            pltpu.VMEM((k_per, WMAX), jnp.float32),
            pltpu.VMEM((k_per, WMAX), jnp.float32),
            pltpu.VMEM((k_per, WMAX), jnp.bfloat16),
            pltpu.VMEM((m, WMAX), jnp.float32),
            pltpu.VMEM((m, WMAX), jnp.float32),
            pltpu.VMEM((m, WMAX), jnp.float32),
            pltpu.VMEM((m, WMAX), jnp.float32),
            pltpu.VMEM((m_out, half_f), jnp.float32),
            pltpu.SemaphoreType.DMA((2,)),
            pltpu.SemaphoreType.DMA((2,)),
            pltpu.SemaphoreType.DMA((N_P,)),
            pltpu.SemaphoreType.DMA((n_chunks,)),
            pltpu.SemaphoreType.DMA((n_chunks,)),
            pltpu.SemaphoreType.DMA((n_chunks,)),
            pltpu.SemaphoreType.DMA((n_chunks,)),
        ],
        compiler_params=pltpu.CompilerParams(
            collective_id=0,
            vmem_limit_bytes=64 * 1024 * 1024,
        ),
    )(x, dy)


# device time: 233635 ns/iter; 1.0413x vs baseline; 1.0413x over previous
import jax
import jax.numpy as jnp
from jax import lax
from jax.experimental import pallas as pl
from jax.experimental.pallas import tpu as pltpu

WMAX = 256
SIZES = [128, 128] + [256] * 14 + [128, 128]
OFFS = [sum(SIZES[:i]) for i in range(len(SIZES))]
N_P = 4
K_SLAB = 512


def kernel(x, dy):
    k_per, m = x.shape
    _, f = dy.shape
    m_out = m // 2
    half_f = f // 2
    n_chunks = len(SIZES)
    assert sum(SIZES) == half_f
    n_slabs = k_per // K_SLAB

    def body(x_ref, dy_ref, out_ref, xb, xl0, xl1, dyv0, dyv1, dyb,
             p0, p1, p2, p3, recv_flat,
             x_sems, dy_sems, st_sems, sx, rx, sy, ry):
        my_x = lax.axis_index("x")
        my_y = lax.axis_index("y")
        other_x = 1 - my_x
        other_y = 1 - my_y
        my_row0 = my_x * m_out
        other_row0 = other_x * m_out

        xl = [xl0, xl1]
        dyv = [dyv0, dyv1]
        pb = [p0, p1, p2, p3]
        ld_objs = [None] * n_chunks
        st_objs = [None] * n_chunks
        rdx_objs = [None] * n_chunks
        rdy_objs = [None] * n_chunks

        def start_load(c):
            ld = pltpu.make_async_copy(
                dy_ref.at[:, pl.ds(my_y * half_f + OFFS[c], SIZES[c])],
                dyv[c % 2].at[:, pl.ds(0, SIZES[c])], dy_sems.at[c % 2])
            ld.start()
            ld_objs[c] = ld

        def finish_chunk(c):
            w = SIZES[c]
            gcol = my_y * half_f + OFFS[c]
            p = pb[c % N_P]
            rdx_objs[c].wait_recv()
            p[pl.ds(my_row0, m_out), pl.ds(0, w)] = (
                p[pl.ds(my_row0, m_out), pl.ds(0, w)]
                + recv_flat[:, pl.ds(OFFS[c], w)])
            rdma_y = pltpu.make_async_remote_copy(
                src_ref=p.at[pl.ds(my_row0, m_out), pl.ds(0, w)],
                dst_ref=out_ref.at[:, pl.ds(gcol, w)],
                send_sem=sy.at[c],
                recv_sem=ry.at[c],
                device_id=(my_x, other_y),
                device_id_type=pl.DeviceIdType.MESH,
            )
            rdma_y.start()
            rdy_objs[c] = rdma_y
            st = pltpu.make_async_copy(
                p.at[pl.ds(my_row0, m_out), pl.ds(0, w)],
                out_ref.at[:, pl.ds(gcol, w)], st_sems.at[c % N_P])
            st.start()
            st_objs[c] = st

        start_load(0)
        x_lds = []
        for i in range(min(2, n_slabs)):
            ld = pltpu.make_async_copy(
                x_ref.at[pl.ds(i * K_SLAB, K_SLAB), :], xl[i % 2],
                x_sems.at[i % 2])
            ld.start()
            x_lds.append(ld)
        for i in range(n_slabs):
            x_lds[i].wait()
            if i + 2 < n_slabs:
                ld = pltpu.make_async_copy(
                    x_ref.at[pl.ds((i + 2) * K_SLAB, K_SLAB), :],
                    xl[(i + 2) % 2], x_sems.at[(i + 2) % 2])
                ld.start()
                x_lds.append(ld)
            xb[pl.ds(i * K_SLAB, K_SLAB), :] = (
                xl[i % 2][...].astype(jnp.bfloat16))

        barrier = pltpu.get_barrier_semaphore()
        pl.semaphore_signal(barrier, inc=1, device_id=(other_x, my_y),
                            device_id_type=pl.DeviceIdType.MESH)
        pl.semaphore_signal(barrier, inc=1, device_id=(my_x, other_y),
                            device_id_type=pl.DeviceIdType.MESH)
        pl.semaphore_wait(barrier, 2)

        for c in range(n_chunks):
            w = SIZES[c]
            p = pb[c % N_P]
            ld_objs[c].wait()
            dyb[:, pl.ds(0, w)] = (
                dyv[c % 2][:, pl.ds(0, w)].astype(jnp.bfloat16))
            if c + 1 < n_chunks:
                start_load(c + 1)

            if c >= N_P:
                rdx_objs[c - N_P].wait_send()
                rdy_objs[c - N_P].wait_send()
                st_objs[c - N_P].wait()

            p[:, pl.ds(0, w)] = lax.dot_general(
                xb[...], dyb[:, pl.ds(0, w)], (((0,), (0,)), ((), ())),
                preferred_element_type=jnp.float32)

            rdma_x = pltpu.make_async_remote_copy(
                src_ref=p.at[pl.ds(other_row0, m_out), pl.ds(0, w)],
                dst_ref=recv_flat.at[:, pl.ds(OFFS[c], w)],
                send_sem=sx.at[c],
                recv_sem=rx.at[c],
                device_id=(other_x, my_y),
                device_id_type=pl.DeviceIdType.MESH,
            )
            rdma_x.start()
            rdx_objs[c] = rdma_x

            if c >= 1:
                finish_chunk(c - 1)

        finish_chunk(n_chunks - 1)

        for c in range(max(n_chunks - N_P, 0), n_chunks):
            rdx_objs[c].wait_send()
            rdy_objs[c].wait_send()
            st_objs[c].wait()
        for c in range(n_chunks):
            rdy_objs[c].wait_recv()

    return pl.pallas_call(
        body,
        out_shape=jax.ShapeDtypeStruct((m_out, f), jnp.float32),
        in_specs=[
            pl.BlockSpec(memory_space=pl.MemorySpace.ANY),
            pl.BlockSpec(memory_space=pl.MemorySpace.ANY),
        ],
        out_specs=pl.BlockSpec(memory_space=pl.MemorySpace.ANY),
        scratch_shapes=[
            pltpu.VMEM((k_per, m), jnp.bfloat16),
            pltpu.VMEM((K_SLAB, m), jnp.float32),
            pltpu.VMEM((K_SLAB, m), jnp.float32),
            pltpu.VMEM((k_per, WMAX), jnp.float32),
            pltpu.VMEM((k_per, WMAX), jnp.float32),
            pltpu.VMEM((k_per, WMAX), jnp.bfloat16),
            pltpu.VMEM((m, WMAX), jnp.float32),
            pltpu.VMEM((m, WMAX), jnp.float32),
            pltpu.VMEM((m, WMAX), jnp.float32),
            pltpu.VMEM((m, WMAX), jnp.float32),
            pltpu.VMEM((m_out, half_f), jnp.float32),
            pltpu.SemaphoreType.DMA((2,)),
            pltpu.SemaphoreType.DMA((2,)),
            pltpu.SemaphoreType.DMA((N_P,)),
            pltpu.SemaphoreType.DMA((n_chunks,)),
            pltpu.SemaphoreType.DMA((n_chunks,)),
            pltpu.SemaphoreType.DMA((n_chunks,)),
            pltpu.SemaphoreType.DMA((n_chunks,)),
        ],
        compiler_params=pltpu.CompilerParams(
            collective_id=0,
            vmem_limit_bytes=64 * 1024 * 1024,
        ),
    )(x, dy)


# device time: 150796 ns/iter; 1.6133x vs baseline; 1.5493x over previous
import jax
import jax.numpy as jnp
from jax import lax
from jax.experimental import pallas as pl
from jax.experimental.pallas import tpu as pltpu

WMAX = 256
SIZES = [256] * 16
OFFS = [sum(SIZES[:i]) for i in range(len(SIZES))]
N_P = 4
K_SLAB = 512
Y_LAG = 3


def kernel(x, dy):
    k_per, m = x.shape
    _, f = dy.shape
    m_out = m // 2
    half_f = f // 2
    n_chunks = len(SIZES)
    assert sum(SIZES) == half_f
    n_slabs = k_per // K_SLAB

    def body(x_ref, dy_ref, out_ref, xb, xl0, xl1, dyv0, dyv1, dyb,
             p0, p1, p2, p3, sb, yb, recv_flat, yrecv, ytmp,
             x_sems, dy_sems, st_sems, yst_sems, sx, rx, sy, ry):
        my_x = lax.axis_index("x")
        my_y = lax.axis_index("y")
        other_x = 1 - my_x
        other_y = 1 - my_y
        my_row0 = my_x * m_out
        other_row0 = other_x * m_out

        xl = [xl0, xl1]
        dyv = [dyv0, dyv1]
        pb = [p0, p1, p2, p3]
        ld_objs = [None] * n_chunks
        st_objs = [None] * n_chunks
        yst_objs = [None] * n_chunks
        rdx_objs = [None] * n_chunks
        rdy_objs = [None] * n_chunks

        def start_load(c):
            ld = pltpu.make_async_copy(
                dy_ref.at[:, pl.ds(my_y * half_f + OFFS[c], SIZES[c])],
                dyv[c % 2].at[:, pl.ds(0, SIZES[c])], dy_sems.at[c % 2])
            ld.start()
            ld_objs[c] = ld

        def finish_chunk(c):
            w = SIZES[c]
            gcol = my_y * half_f + OFFS[c]
            p = pb[c % N_P]
            rdx_objs[c].wait_recv()
            p[pl.ds(my_row0, m_out), pl.ds(0, w)] = (
                p[pl.ds(my_row0, m_out), pl.ds(0, w)]
                + recv_flat[:, pl.ds(OFFS[c], w)].astype(jnp.float32))
            if c >= N_P:
                rdy_objs[c - N_P].wait_send()
            yb[c % N_P, :, pl.ds(0, w)] = (
                p[pl.ds(my_row0, m_out), pl.ds(0, w)].astype(jnp.bfloat16))
            rdma_y = pltpu.make_async_remote_copy(
                src_ref=yb.at[c % N_P, :, pl.ds(0, w)],
                dst_ref=yrecv.at[:, pl.ds(OFFS[c], w)],
                send_sem=sy.at[c],
                recv_sem=ry.at[c],
                device_id=(my_x, other_y),
                device_id_type=pl.DeviceIdType.MESH,
            )
            rdma_y.start()
            rdy_objs[c] = rdma_y
            st = pltpu.make_async_copy(
                p.at[pl.ds(my_row0, m_out), pl.ds(0, w)],
                out_ref.at[:, pl.ds(gcol, w)], st_sems.at[c % N_P])
            st.start()
            st_objs[c] = st

        def service_y(c):
            w = SIZES[c]
            gcol = other_y * half_f + OFFS[c]
            rdy_objs[c].wait_recv()
            if c >= N_P:
                yst_objs[c - N_P].wait()
            ytmp[c % N_P, :, pl.ds(0, w)] = (
                yrecv[:, pl.ds(OFFS[c], w)].astype(jnp.float32))
            yst = pltpu.make_async_copy(
                ytmp.at[c % N_P, :, pl.ds(0, w)],
                out_ref.at[:, pl.ds(gcol, w)], yst_sems.at[c % N_P])
            yst.start()
            yst_objs[c] = yst

        start_load(0)
        x_lds = []
        for i in range(min(2, n_slabs)):
            ld = pltpu.make_async_copy(
                x_ref.at[pl.ds(i * K_SLAB, K_SLAB), :], xl[i % 2],
                x_sems.at[i % 2])
            ld.start()
            x_lds.append(ld)
        for i in range(n_slabs):
            x_lds[i].wait()
            if i + 2 < n_slabs:
                ld = pltpu.make_async_copy(
                    x_ref.at[pl.ds((i + 2) * K_SLAB, K_SLAB), :],
                    xl[(i + 2) % 2], x_sems.at[(i + 2) % 2])
                ld.start()
                x_lds.append(ld)
            xb[pl.ds(i * K_SLAB, K_SLAB), :] = (
                xl[i % 2][...].astype(jnp.bfloat16))

        barrier = pltpu.get_barrier_semaphore()
        pl.semaphore_signal(barrier, inc=1, device_id=(other_x, my_y),
                            device_id_type=pl.DeviceIdType.MESH)
        pl.semaphore_signal(barrier, inc=1, device_id=(my_x, other_y),
                            device_id_type=pl.DeviceIdType.MESH)
        pl.semaphore_wait(barrier, 2)

        for c in range(n_chunks):
            w = SIZES[c]
            p = pb[c % N_P]
            ld_objs[c].wait()
            dyb[:, pl.ds(0, w)] = (
                dyv[c % 2][:, pl.ds(0, w)].astype(jnp.bfloat16))
            if c + 1 < n_chunks:
                start_load(c + 1)

            if c >= N_P:
                st_objs[c - N_P].wait()

            p[:, pl.ds(0, w)] = lax.dot_general(
                xb[...], dyb[:, pl.ds(0, w)], (((0,), (0,)), ((), ())),
                preferred_element_type=jnp.float32)

            if c >= N_P:
                rdx_objs[c - N_P].wait_send()
            sb[c % N_P, :, pl.ds(0, w)] = (
                p[pl.ds(other_row0, m_out), pl.ds(0, w)].astype(
                    jnp.bfloat16))
            rdma_x = pltpu.make_async_remote_copy(
                src_ref=sb.at[c % N_P, :, pl.ds(0, w)],
                dst_ref=recv_flat.at[:, pl.ds(OFFS[c], w)],
                send_sem=sx.at[c],
                recv_sem=rx.at[c],
                device_id=(other_x, my_y),
                device_id_type=pl.DeviceIdType.MESH,
            )
            rdma_x.start()
            rdx_objs[c] = rdma_x

            if c >= 1:
                finish_chunk(c - 1)
            if c >= Y_LAG:
                service_y(c - Y_LAG)

        finish_chunk(n_chunks - 1)
        for c in range(max(n_chunks - Y_LAG, 0), n_chunks):
            service_y(c)

        for c in range(max(n_chunks - N_P, 0), n_chunks):
            rdx_objs[c].wait_send()
            rdy_objs[c].wait_send()
            st_objs[c].wait()
            yst_objs[c].wait()

    return pl.pallas_call(
        body,
        out_shape=jax.ShapeDtypeStruct((m_out, f), jnp.float32),
        in_specs=[
            pl.BlockSpec(memory_space=pl.MemorySpace.ANY),
            pl.BlockSpec(memory_space=pl.MemorySpace.ANY),
        ],
        out_specs=pl.BlockSpec(memory_space=pl.MemorySpace.ANY),
        scratch_shapes=[
            pltpu.VMEM((k_per, m), jnp.bfloat16),
            pltpu.VMEM((K_SLAB, m), jnp.float32),
            pltpu.VMEM((K_SLAB, m), jnp.float32),
            pltpu.VMEM((k_per, WMAX), jnp.float32),
            pltpu.VMEM((k_per, WMAX), jnp.float32),
            pltpu.VMEM((k_per, WMAX), jnp.bfloat16),
            pltpu.VMEM((m, WMAX), jnp.float32),
            pltpu.VMEM((m, WMAX), jnp.float32),
            pltpu.VMEM((m, WMAX), jnp.float32),
            pltpu.VMEM((m, WMAX), jnp.float32),
            pltpu.VMEM((N_P, m_out, WMAX), jnp.bfloat16),
            pltpu.VMEM((N_P, m_out, WMAX), jnp.bfloat16),
            pltpu.VMEM((m_out, half_f), jnp.bfloat16),
            pltpu.VMEM((m_out, half_f), jnp.bfloat16),
            pltpu.VMEM((N_P, m_out, WMAX), jnp.float32),
            pltpu.SemaphoreType.DMA((2,)),
            pltpu.SemaphoreType.DMA((2,)),
            pltpu.SemaphoreType.DMA((N_P,)),
            pltpu.SemaphoreType.DMA((N_P,)),
            pltpu.SemaphoreType.DMA((n_chunks,)),
            pltpu.SemaphoreType.DMA((n_chunks,)),
            pltpu.SemaphoreType.DMA((n_chunks,)),
            pltpu.SemaphoreType.DMA((n_chunks,)),
        ],
        compiler_params=pltpu.CompilerParams(
            collective_id=0,
            vmem_limit_bytes=64 * 1024 * 1024,
        ),
    )(x, dy)
